# baseline (device time: 120982 ns/iter reference)
import jax
import jax.numpy as jnp
from jax import lax
from jax.experimental import pallas as pl
from jax.experimental.pallas import tpu as pltpu

N_DEV = 16
M = 4096
N = 1024
HN = N // 2
QW = 256
PCH = 1024
QCH = 256
S1 = 2
SR1 = PCH // S1

S2 = 2
QR = QCH // S2

NSEM = 24


def kernel(x):
    def body(
        x_ref,
        out_ref,
        p1s_f, p1r_f, p1s_b, p1r_b,
        res1_f, res1_b,
        p2s_f, p2r_f, p2s_b, p2r_b,
        red_f, red_b,
        resp_f, resp_b,
        p3_f, p3_b,
        p4_f, p4_b,
        ssem_f, rsem_f, ssem_b, rsem_b,
    ):
        my = lax.axis_index("i")
        z = my // 4
        j = lax.rem(my, 4)
        succ_p = 4 * z + lax.rem(j + 1, 4)
        pred_p = 4 * z + lax.rem(j + 3, 4)
        succ_c = 4 * lax.rem(z + 1, 4) + j
        pred_c = 4 * lax.rem(z + 3, 4) + j
        P = lax.rem(j + 1, 4)
        Pb = lax.rem(j + 3, 4)
        Q = lax.rem(z + 1, 4)
        Qb = lax.rem(z + 3, 4)

        def rdma(src, dst, sl, sem_i, fwd, column):
            if fwd:
                tgt, ssem, rsem = (succ_c if column else succ_p), ssem_f, rsem_f
            else:
                tgt, ssem, rsem = (pred_c if column else pred_p), ssem_b, rsem_b
            return pltpu.make_async_remote_copy(
                src_ref=src,
                dst_ref=dst,
                send_sem=ssem.at[sl, sem_i],
                recv_sem=rsem.at[sl, sem_i],
                device_id=(tgt,),
                device_id_type=pl.DeviceIdType.MESH,
            )

        def p1_rdma(sl, h, s, fwd):
            a, b = (p1s_f, p1r_f) if fwd else (p1s_b, p1r_b)
            return rdma(a.at[sl, h, s], b.at[sl, h, s], sl, h * 2 + s, fwd, False)

        def p2_rdma(sl, h, t, fwd):
            a, b = (p2s_f, p2r_f) if fwd else (p2s_b, p2r_b)
            return rdma(
                a.at[sl, h, t], b.at[sl, h, t], sl, 6 + h * 2 + t, fwd, True
            )

        def p3_rdma(sl, h, t, fwd):
            p3, red = (p3_f, red_f) if fwd else (p3_b, red_b)
            src = (
                red.at[sl, t * QR : (t + 1) * QR]
                if h == 0
                else p3.at[sl, h - 1, t]
            )
            return rdma(src, p3.at[sl, h, t], sl, 12 + h * 2 + t, fwd, True)

        def p4_rdma(sl, h, s, fwd):
            p4, resp = (p4_f, resp_f) if fwd else (p4_b, resp_b)
            if h == 0:
                src = resp.at[sl, 2 * s : 2 * s + 2]
            else:
                src = p4.at[sl, h - 1, s]
            return rdma(src, p4.at[sl, h, s], sl, 18 + h * 2 + s, fwd, False)

        def xsub(c, s, fwd, sl):
            c0 = (0 if fwd else HN) + sl * QW
            return x_ref[
                pl.ds(c * PCH + s * SR1, SR1), c0 : c0 + QW
            ].astype(jnp.bfloat16)

        def ocols(fwd, sl):
            c0 = (0 if fwd else HN) + sl * QW
            return slice(c0, c0 + QW)

        barrier_sem = pltpu.get_barrier_semaphore()
        for nbr in (succ_p, pred_p, succ_c, pred_c):
            pl.semaphore_signal(
                barrier_sem, inc=1, device_id=(nbr,),
                device_id_type=pl.DeviceIdType.MESH,
            )
        pl.semaphore_wait(barrier_sem, 4)

        def p1_seed(sl):
            for s in range(S1):
                p1s_f[sl, 0, s, :, :] = xsub(j, s, True, sl)
                p1_rdma(sl, 0, s, True).start()
                p1s_b[sl, 0, s, :, :] = xsub(j, s, False, sl)
                p1_rdma(sl, 0, s, False).start()

        def p1_hop(sl, h):
            cf = lax.rem(j - (h + 1) + 8, 4)
            cb = lax.rem(j + h + 1, 4)
            for s in range(S1):
                p1_rdma(sl, h, s, True).wait_recv()
                if h < 2:
                    p1s_f[sl, h + 1, s, :, :] = (
                        p1r_f[sl, h, s] + xsub(cf, s, True, sl)
                    )
                    p1_rdma(sl, h + 1, s, True).start()
                else:
                    res1_f[sl, 2 * s : 2 * s + 2, :, :] = (
                        p1r_f[sl, h, s] + xsub(cf, s, True, sl)
                    ).reshape(2, QCH, QW)
                p1_rdma(sl, h, s, False).wait_recv()
                if h < 2:
                    p1s_b[sl, h + 1, s, :, :] = (
                        p1r_b[sl, h, s] + xsub(cb, s, False, sl)
                    )
                    p1_rdma(sl, h + 1, s, False).start()
                else:
                    res1_b[sl, 2 * s : 2 * s + 2, :, :] = (
                        p1r_b[sl, h, s] + xsub(cb, s, False, sl)
                    ).reshape(2, QCH, QW)

        def p2_seed(sl):
            for t in range(S2):
                rows = slice(t * QR, (t + 1) * QR)
                p2s_f[sl, 0, t, :, :] = res1_f[sl, z, rows]
                p2_rdma(sl, 0, t, True).start()
                p2s_b[sl, 0, t, :, :] = res1_b[sl, z, rows]
                p2_rdma(sl, 0, t, False).start()

        def p2_hop(sl, h):
            qf = lax.rem(z - (h + 1) + 8, 4)
            qb = lax.rem(z + h + 1, 4)
            for t in range(S2):
                rows = slice(t * QR, (t + 1) * QR)
                p2_rdma(sl, h, t, True).wait_recv()
                if h < 2:
                    p2s_f[sl, h + 1, t, :, :] = (
                        p2r_f[sl, h, t] + res1_f[sl, qf, rows]
                    )
                    p2_rdma(sl, h + 1, t, True).start()
                else:
                    red_f[sl, rows, :] = p2r_f[sl, h, t] + res1_f[sl, qf, rows]
                    p3_rdma(sl, 0, t, True).start()
                p2_rdma(sl, h, t, False).wait_recv()
                if h < 2:
                    p2s_b[sl, h + 1, t, :, :] = (
                        p2r_b[sl, h, t] + res1_b[sl, qb, rows]
                    )
                    p2_rdma(sl, h + 1, t, False).start()
                else:
                    red_b[sl, rows, :] = p2r_b[sl, h, t] + res1_b[sl, qb, rows]
                    p3_rdma(sl, 0, t, False).start()
            if h == 2:
                resp_f[sl, Q, :, :] = red_f[sl]
                out_ref[pl.ds(P * PCH + Q * QCH, QCH), ocols(True, sl)] = (
                    red_f[sl].astype(jnp.float32)
                )
                resp_b[sl, Qb, :, :] = red_b[sl]
                out_ref[pl.ds(Pb * PCH + Qb * QCH, QCH), ocols(False, sl)] = (
                    red_b[sl].astype(jnp.float32)
                )

        def p3_hop(sl, h):
            qf = lax.rem(z - h + 8, 4)
            qb = lax.rem(z + h, 4)
            for t in range(S2):
                p3_rdma(sl, h, t, True).wait_recv()
                if h < 2:
                    p3_rdma(sl, h + 1, t, True).start()
                p3_rdma(sl, h, t, False).wait_recv()
                if h < 2:
                    p3_rdma(sl, h + 1, t, False).start()
            resp_f[sl, qf, :, :] = p3_f[sl, h].reshape(QCH, QW)
            out_ref[pl.ds(P * PCH + qf * QCH, QCH), ocols(True, sl)] = (
                p3_f[sl, h].astype(jnp.float32).reshape(QCH, QW)
            )
            resp_b[sl, qb, :, :] = p3_b[sl, h].reshape(QCH, QW)
            out_ref[pl.ds(Pb * PCH + qb * QCH, QCH), ocols(False, sl)] = (
                p3_b[sl, h].astype(jnp.float32).reshape(QCH, QW)
            )

        def p4_seed(sl):
            for s in range(S1):
                p4_rdma(sl, 0, s, True).start()
                p4_rdma(sl, 0, s, False).start()

        def p4_hop(sl, h):
            chf = lax.rem(j - h + 8, 4)
            chb = lax.rem(j + h, 4)
            for s in range(S1):
                p4_rdma(sl, h, s, True).wait_recv()
                if h < 2:
                    p4_rdma(sl, h + 1, s, True).start()
                p4_rdma(sl, h, s, False).wait_recv()
                if h < 2:
                    p4_rdma(sl, h + 1, s, False).start()
            out_ref[pl.ds(chf * PCH, PCH), ocols(True, sl)] = (
                p4_f[sl, h].astype(jnp.float32).reshape(PCH, QW)
            )
            out_ref[pl.ds(chb * PCH, PCH), ocols(False, sl)] = (
                p4_b[sl, h].astype(jnp.float32).reshape(PCH, QW)
            )

        p1_seed(0)
        for h in range(3):
            p1_hop(0, h)
        p2_seed(0)
        p1_seed(1)
        for h in range(3):
            p2_hop(0, h)
            p1_hop(1, h)
        p2_seed(1)
        for h in range(3):
            p3_hop(0, h)
            p2_hop(1, h)
        p4_seed(0)
        for h in range(3):
            p4_hop(0, h)
            p3_hop(1, h)
        p4_seed(1)
        for h in range(3):
            p4_hop(1, h)

        for sl in range(2):
            for h in range(3):
                for s in range(S1):
                    p1_rdma(sl, h, s, True).wait_send()
                    p1_rdma(sl, h, s, False).wait_send()
                    p4_rdma(sl, h, s, True).wait_send()
                    p4_rdma(sl, h, s, False).wait_send()
                for t in range(S2):
                    p2_rdma(sl, h, t, True).wait_send()
                    p2_rdma(sl, h, t, False).wait_send()
                    p3_rdma(sl, h, t, True).wait_send()
                    p3_rdma(sl, h, t, False).wait_send()

    bf = jnp.bfloat16
    return pl.pallas_call(
        body,
        out_shape=jax.ShapeDtypeStruct((M, N), jnp.float32),
        in_specs=[pl.BlockSpec(memory_space=pltpu.VMEM)],
        out_specs=pl.BlockSpec(memory_space=pltpu.VMEM),
        scratch_shapes=[
            pltpu.VMEM((2, 3, S1, SR1, QW), bf),
            pltpu.VMEM((2, 3, S1, SR1, QW), bf),
            pltpu.VMEM((2, 3, S1, SR1, QW), bf),
            pltpu.VMEM((2, 3, S1, SR1, QW), bf),
            pltpu.VMEM((2, 4, QCH, QW), bf),
            pltpu.VMEM((2, 4, QCH, QW), bf),
            pltpu.VMEM((2, 3, S2, QR, QW), bf),
            pltpu.VMEM((2, 3, S2, QR, QW), bf),
            pltpu.VMEM((2, 3, S2, QR, QW), bf),
            pltpu.VMEM((2, 3, S2, QR, QW), bf),
            pltpu.VMEM((2, QCH, QW), bf),
            pltpu.VMEM((2, QCH, QW), bf),
            pltpu.VMEM((2, 4, QCH, QW), bf),
            pltpu.VMEM((2, 4, QCH, QW), bf),
            pltpu.VMEM((2, 3, S2, QR, QW), bf),
            pltpu.VMEM((2, 3, S2, QR, QW), bf),
            pltpu.VMEM((2, 3, S1, 2, QCH, QW), bf),
            pltpu.VMEM((2, 3, S1, 2, QCH, QW), bf),
            pltpu.SemaphoreType.DMA((2, NSEM)),
            pltpu.SemaphoreType.DMA((2, NSEM)),
            pltpu.SemaphoreType.DMA((2, NSEM)),
            pltpu.SemaphoreType.DMA((2, NSEM)),
        ],
        compiler_params=pltpu.CompilerParams(
            vmem_limit_bytes=100 * 1024 * 1024,
            collective_id=0,
        ),
    )(x)


# device time: 118402 ns/iter; 1.0218x vs baseline; 1.0218x over previous
import jax
import jax.numpy as jnp
from jax import lax
from jax.experimental import pallas as pl
from jax.experimental.pallas import tpu as pltpu

N_DEV = 16
M = 4096
N = 1024
CH = M // N_DEV
HN = N // 2
SUB = 4
SR = CH // SUB
NH = N_DEV - 1

_POS = [0, 1, 8, 9, 15, 2, 7, 10, 14, 3, 6, 11, 13, 4, 5, 12]
_SUCC = [1, 5, 3, 7, 0, 9, 2, 11, 4, 13, 6, 15, 8, 14, 10, 12]
_PRED = [4, 0, 6, 2, 8, 1, 10, 3, 12, 5, 14, 7, 15, 9, 13, 11]


def kernel(x):
    def body(
        x_ref,
        out_ref,
        rs_send_f,
        rs_recv_f,
        rs_send_b,
        rs_recv_b,
        ag_f,
        ag_b,
        rs_ssem_f,
        rs_rsem_f,
        rs_ssem_b,
        rs_rsem_b,
        ag_ssem_f,
        ag_rsem_f,
        ag_ssem_b,
        ag_rsem_b,
    ):
        my = lax.axis_index("i")

        def lut(table):
            v = jnp.int32(table[0])
            for k in range(1, N_DEV):
                v = jnp.where(my == k, jnp.int32(table[k]), v)
            return v

        r = lut(_POS)
        right = lut(_SUCC)
        left = lut(_PRED)

        def xsub(c, s, fwd):
            cols = slice(0, HN) if fwd else slice(HN, N)
            return x_ref[pl.ds(c * CH + s * SR, SR), cols].astype(jnp.bfloat16)

        def rs_rdma(h, s, fwd):
            if fwd:
                return pltpu.make_async_remote_copy(
                    src_ref=rs_send_f.at[h, s],
                    dst_ref=rs_recv_f.at[h, s],
                    send_sem=rs_ssem_f.at[h * SUB + s],
                    recv_sem=rs_rsem_f.at[h * SUB + s],
                    device_id=(right,),
                    device_id_type=pl.DeviceIdType.MESH,
                )
            return pltpu.make_async_remote_copy(
                src_ref=rs_send_b.at[h, s],
                dst_ref=rs_recv_b.at[h, s],
                send_sem=rs_ssem_b.at[h * SUB + s],
                recv_sem=rs_rsem_b.at[h * SUB + s],
                device_id=(left,),
                device_id_type=pl.DeviceIdType.MESH,
            )

        def ag_rdma(h, s, fwd):
            if fwd:
                return pltpu.make_async_remote_copy(
                    src_ref=ag_f.at[NH if h == 0 else h - 1, s],
                    dst_ref=ag_f.at[h, s],
                    send_sem=ag_ssem_f.at[h * SUB + s],
                    recv_sem=ag_rsem_f.at[h * SUB + s],
                    device_id=(right,),
                    device_id_type=pl.DeviceIdType.MESH,
                )
            return pltpu.make_async_remote_copy(
                src_ref=ag_b.at[NH if h == 0 else h - 1, s],
                dst_ref=ag_b.at[h, s],
                send_sem=ag_ssem_b.at[h * SUB + s],
                recv_sem=ag_rsem_b.at[h * SUB + s],
                device_id=(left,),
                device_id_type=pl.DeviceIdType.MESH,
            )

        barrier_sem = pltpu.get_barrier_semaphore()
        pl.semaphore_signal(
            barrier_sem, inc=1, device_id=(left,),
            device_id_type=pl.DeviceIdType.MESH,
        )
        pl.semaphore_signal(
            barrier_sem, inc=1, device_id=(right,),
            device_id_type=pl.DeviceIdType.MESH,
        )
        pl.semaphore_wait(barrier_sem, 2)

        for s in range(SUB):
            rs_send_f[0, s, :, :] = xsub(r, s, True)
            rs_rdma(0, s, True).start()
            rs_send_b[0, s, :, :] = xsub(r, s, False)
            rs_rdma(0, s, False).start()

        for h in range(NH):
            cf = lax.rem(r - (h + 1) + 2 * N_DEV, N_DEV)
            cb = lax.rem(r + h + 1, N_DEV)
            for s in range(SUB):
                rs_rdma(h, s, True).wait_recv()
                if h < NH - 1:
                    rs_send_f[h + 1, s, :, :] = rs_recv_f[h, s] + xsub(cf, s, True)
                    rs_rdma(h + 1, s, True).start()
                else:
                    ag_f[NH, s, :, :] = rs_recv_f[h, s] + xsub(cf, s, True)
                    ag_rdma(0, s, True).start()
                rs_rdma(h, s, False).wait_recv()
                if h < NH - 1:
                    rs_send_b[h + 1, s, :, :] = rs_recv_b[h, s] + xsub(cb, s, False)
                    rs_rdma(h + 1, s, False).start()
                else:
                    ag_b[NH, s, :, :] = rs_recv_b[h, s] + xsub(cb, s, False)
                    ag_rdma(0, s, False).start()

        own_f = lax.rem(r + 1, N_DEV)
        own_b = lax.rem(r + N_DEV - 1, N_DEV)
        out_ref[pl.ds(own_f * CH, CH), 0:HN] = (
            ag_f[NH].astype(jnp.float32).reshape(CH, HN)
        )
        out_ref[pl.ds(own_b * CH, CH), HN:N] = (
            ag_b[NH].astype(jnp.float32).reshape(CH, HN)
        )

        for h in range(NH):
            cf = lax.rem(r - h + 2 * N_DEV, N_DEV)
            cb = lax.rem(r + h, N_DEV)
            for s in range(SUB):
                ag_rdma(h, s, True).wait_recv()
                if h < NH - 1:
                    ag_rdma(h + 1, s, True).start()
                ag_rdma(h, s, False).wait_recv()
                if h < NH - 1:
                    ag_rdma(h + 1, s, False).start()
            out_ref[pl.ds(cf * CH, CH), 0:HN] = (
                ag_f[h].astype(jnp.float32).reshape(CH, HN)
            )
            out_ref[pl.ds(cb * CH, CH), HN:N] = (
                ag_b[h].astype(jnp.float32).reshape(CH, HN)
            )

        for h in range(NH):
            for s in range(SUB):
                rs_rdma(h, s, True).wait_send()
                rs_rdma(h, s, False).wait_send()
                ag_rdma(h, s, True).wait_send()
                ag_rdma(h, s, False).wait_send()

    rs_shape = (NH, SUB, SR, HN)
    ag_shape = (N_DEV, SUB, SR, HN)
    nsem = NH * SUB
    return pl.pallas_call(
        body,
        out_shape=jax.ShapeDtypeStruct((M, N), jnp.float32),
        in_specs=[pl.BlockSpec(memory_space=pltpu.VMEM)],
        out_specs=pl.BlockSpec(memory_space=pltpu.VMEM),
        scratch_shapes=[
            pltpu.VMEM(rs_shape, jnp.bfloat16),
            pltpu.VMEM(rs_shape, jnp.bfloat16),
            pltpu.VMEM(rs_shape, jnp.bfloat16),
            pltpu.VMEM(rs_shape, jnp.bfloat16),
            pltpu.VMEM(ag_shape, jnp.bfloat16),
            pltpu.VMEM(ag_shape, jnp.bfloat16),
            pltpu.SemaphoreType.DMA((nsem,)),
            pltpu.SemaphoreType.DMA((nsem,)),
            pltpu.SemaphoreType.DMA((nsem,)),
            pltpu.SemaphoreType.DMA((nsem,)),
            pltpu.SemaphoreType.DMA((nsem,)),
            pltpu.SemaphoreType.DMA((nsem,)),
            pltpu.SemaphoreType.DMA((nsem,)),
            pltpu.SemaphoreType.DMA((nsem,)),
        ],
        compiler_params=pltpu.CompilerParams(
            vmem_limit_bytes=100 * 1024 * 1024,
            collective_id=0,
        ),
    )(x)
